# baseline (speedup 1.0000x reference)
import functools

import jax
import jax.numpy as jnp
import numpy as np
from jax import lax
from jax.experimental import pallas as pl
from jax.experimental.pallas import tpu as pltpu
from jax.experimental.pallas import tpu_sc as plsc

NT = ('author', 'paper')
ET = (('author', 'writes', 'paper'), ('paper', 'rev_writes', 'author'))
N = 10000
HID = 512
H = 8
D = 64
E = 160000

NCORES = 2
NSUB = 16
NWORKERS = NCORES * NSUB
CHUNK = 64
NCHUNKS = E // CHUNK
EXTRA = NCHUNKS - (NCHUNKS // NWORKERS) * NWORKERS
NPAD = 10240
ROWS_PER_TILE = NPAD // NSUB
NEG = np.float32(-3.0e38)

_sc_mesh = plsc.VectorSubcoreMesh(core_axis_name="c", subcore_axis_name="s")


def _ekey(et):
    return '__'.join(et)


def _mm_body(x_ref, w_ref, b_ref, o_ref, *, act_in, act_out):
    x = x_ref[...]
    if act_in == 'gelu':
        x = jax.nn.gelu(x)
    y = jnp.dot(x, w_ref[...], preferred_element_type=jnp.float32) + b_ref[...]
    if act_out == 'relu':
        y = jnp.maximum(y, 0.0)
    o_ref[...] = y


def _mm_res_body(x_ref, w_ref, b_ref, r_ref, o_ref, *, act_in, act_out):
    x = x_ref[...]
    if act_in == 'gelu':
        x = jax.nn.gelu(x)
    y = jnp.dot(x, w_ref[...], preferred_element_type=jnp.float32) + b_ref[...]
    o_ref[...] = y + r_ref[...]


def _matmul(x, w, b, res=None, act_in=None, act_out=None, block_m=1000):
    m, k = x.shape
    k2, n = w.shape
    assert k == k2 and m % block_m == 0
    grid = (m // block_m,)
    b2 = jnp.broadcast_to(b.reshape(1, n), (8, n))
    in_specs = [
        pl.BlockSpec((block_m, k), lambda i: (i, 0)),
        pl.BlockSpec((k, n), lambda i: (0, 0)),
        pl.BlockSpec((8, n), lambda i: (0, 0)),
    ]
    args = [x, w, b2]
    if res is None:
        body = functools.partial(_mm_body, act_in=act_in, act_out=act_out)
    else:
        body = functools.partial(_mm_res_body, act_in=act_in, act_out=act_out)
        in_specs.append(pl.BlockSpec((block_m, n), lambda i: (i, 0)))
        args.append(res)
    return pl.pallas_call(
        lambda *refs: body(refs[0], refs[1], refs[2][0:1], *refs[3:]),
        grid=grid,
        in_specs=in_specs,
        out_specs=pl.BlockSpec((block_m, n), lambda i: (i, 0)),
        out_shape=jax.ShapeDtypeStruct((m, n), jnp.float32),
    )(*args)


def _combine_rel(w, bvec, rel):
    bd = jax.scipy.linalg.block_diag if False else None
    eye = jnp.eye(H, dtype=jnp.float32)
    blk = jnp.einsum('hij,hg->higj', rel, eye).reshape(HID, HID)
    stack = jnp.concatenate([w, bvec.reshape(1, -1),
                             jnp.zeros((7, HID), jnp.float32)], axis=0)
    out = _matmul(stack, blk, jnp.zeros((HID,), jnp.float32), block_m=520)
    return out[:HID], out[HID]


def _ntiles_chunks(wid):
    return jnp.where(wid < EXTRA, NCHUNKS // NWORKERS + 1, NCHUNKS // NWORKERS)


_GDN = lax.GatherDimensionNumbers(
    offset_dims=(), collapsed_slice_dims=(0,), start_index_map=(0,))


def _lane_perm(x, idx):
    return lax.gather(x, idx[:, None], _GDN, slice_sizes=(1,),
                      mode=lax.GatherScatterMode.PROMISE_IN_BOUNDS)


def _bf_sum(x):
    lanes = lax.iota(jnp.int32, 16)
    for s in (1, 2, 4, 8):
        x = x + _lane_perm(x, lanes ^ s)
    return x


def _bf_max(x):
    lanes = lax.iota(jnp.int32, 16)
    for s in (1, 2, 4, 8):
        x = jnp.maximum(x, _lane_perm(x, lanes ^ s))
    return x


def _worker_id():
    return lax.axis_index("c") * NSUB + lax.axis_index("s")


LCH = 32
LNCH = E // LCH
LMAIN = (LNCH // NWORKERS) & ~1
LEXTRA = LNCH - LMAIN * NWORKERS


@functools.partial(
    pl.kernel,
    out_type=[jax.ShapeDtypeStruct((E, 16), jnp.float32),
              jax.ShapeDtypeStruct((NWORKERS * 16,), jnp.float32)],
    mesh=_sc_mesh,
    compiler_params=pltpu.CompilerParams(needs_layout_passes=False),
    scratch_types=[
        pltpu.VMEM((LCH,), jnp.int32),
        pltpu.VMEM((LCH,), jnp.int32),
        pltpu.VMEM((LCH,), jnp.int32),
        pltpu.VMEM((LCH,), jnp.int32),
        pltpu.VMEM((LCH, HID // 2), jnp.int32),
        pltpu.VMEM((LCH, HID // 2), jnp.int32),
        pltpu.VMEM((LCH, HID // 2), jnp.int32),
        pltpu.VMEM((LCH, HID // 2), jnp.int32),
        pltpu.VMEM((LCH, 16), jnp.float32),
        pltpu.VMEM((16,), jnp.float32),
        pltpu.VMEM((16,), jnp.float32),
        pltpu.SemaphoreType.DMA,
        pltpu.SemaphoreType.DMA,
        pltpu.SemaphoreType.DMA,
        pltpu.SemaphoreType.DMA,
    ],
)
def _sc_logits(q_hbm, k_hbm, s_hbm, d_hbm, p_hbm, lg_hbm, wmax_hbm,
               sv0, sv1, dv0, dv1, k0, k1, q0, q1, lgv, pv, mxv,
               sk0, sk1, sq0, sq1):
    wid = _worker_id()
    pltpu.sync_copy(p_hbm, pv)
    pvv = pv[...]
    lanes = lax.iota(jnp.int32, 16)
    svs, dvs, ks, qs = (sv0, sv1), (dv0, dv1), (k0, k1), (q0, q1)
    sks, sqs = (sk0, sk1), (sq0, sq1)

    def issue(t, b):
        base = (wid + NWORKERS * t) * LCH
        pltpu.sync_copy(s_hbm.at[pl.ds(base, LCH)], svs[b])
        pltpu.sync_copy(d_hbm.at[pl.ds(base, LCH)], dvs[b])
        pltpu.async_copy(k_hbm.at[svs[b]], ks[b], sks[b])
        pltpu.async_copy(q_hbm.at[dvs[b]], qs[b], sqs[b])

    def compute(t, b, mx):
        base = (wid + NWORKERS * t) * LCH
        pltpu.make_async_copy(k_hbm.at[svs[b]], ks[b], sks[b]).wait()
        pltpu.make_async_copy(q_hbm.at[dvs[b]], qs[b], sqs[b]).wait()
        krows, qrows = ks[b], qs[b]

        def edge_body(e2, mx2):
            for u in range(2):
                e = e2 * 2 + u
                row = jnp.zeros((16,), jnp.float32)
                for h in range(H):
                    acc = None
                    for j in range(D // 32):
                        qa, qb = plsc.unpack(
                            plsc.bitcast(qrows[e, pl.ds(h * (D // 2) + j * 16, 16)],
                                         jnp.bfloat16),
                            format=plsc.PackFormat.INTERLEAVED)
                        ka, kb = plsc.unpack(
                            plsc.bitcast(krows[e, pl.ds(h * (D // 2) + j * 16, 16)],
                                         jnp.bfloat16),
                            format=plsc.PackFormat.INTERLEAVED)
                        term = qa * ka + qb * kb
                        acc = term if acc is None else acc + term
                    row = jnp.where(lanes == h, _bf_sum(acc), row)
                row = row * pvv
                lgv[e, :] = row
                mx2 = jnp.maximum(mx2, row)
            return mx2

        mx = lax.fori_loop(0, LCH // 2, edge_body, mx)
        pltpu.sync_copy(lgv, lg_hbm.at[pl.ds(base, LCH)])
        return mx

    issue(0, 0)
    issue(1, 1)

    def pair_body(t2, mx):
        for b in range(2):
            t = t2 * 2 + b
            mx = compute(t, b, mx)

            @pl.when(t + 2 < LMAIN)
            def _():
                issue(t + 2, b)
        return mx

    mx = lax.fori_loop(0, LMAIN // 2, pair_body,
                       jnp.zeros((16,), jnp.float32))

    mxv[...] = mx

    @pl.when(wid < LEXTRA)
    def _():
        base = (LMAIN * NWORKERS + wid) * LCH
        pltpu.sync_copy(s_hbm.at[pl.ds(base, LCH)], sv0)
        pltpu.sync_copy(d_hbm.at[pl.ds(base, LCH)], dv0)
        pltpu.async_copy(k_hbm.at[sv0], k0, sk0).wait()
        pltpu.async_copy(q_hbm.at[dv0], q0, sq0).wait()

        def edge_body(e2, _2):
            for u in range(2):
                e = e2 * 2 + u
                row = jnp.zeros((16,), jnp.float32)
                for h in range(H):
                    acc = None
                    for j in range(D // 32):
                        qa, qb = plsc.unpack(
                            plsc.bitcast(q0[e, pl.ds(h * (D // 2) + j * 16, 16)],
                                         jnp.bfloat16),
                            format=plsc.PackFormat.INTERLEAVED)
                        ka, kb = plsc.unpack(
                            plsc.bitcast(k0[e, pl.ds(h * (D // 2) + j * 16, 16)],
                                         jnp.bfloat16),
                            format=plsc.PackFormat.INTERLEAVED)
                        term = qa * ka + qb * kb
                        acc = term if acc is None else acc + term
                    row = jnp.where(lanes == h, _bf_sum(acc), row)
                row = row * pvv
                lgv[e, :] = row
                mxv[...] = jnp.maximum(mxv[...], row)
            return 0

        lax.fori_loop(0, LCH // 2, edge_body, 0)
        pltpu.sync_copy(lgv, lg_hbm.at[pl.ds(base, LCH)])

    mxv[...] = _bf_max(mxv[...])
    pltpu.sync_copy(mxv, wmax_hbm.at[pl.ds(wid * 16, 16)])


def _gmax_of(wmv):
    g = wmv[pl.ds(0, 16)]
    for i in range(1, NWORKERS):
        g = jnp.maximum(g, wmv[pl.ds(i * 16, 16)])
    return _bf_max(g)


def _gmax_mem(wmv, tmp):
    g = wmv[pl.ds(0, 16)]
    for i in range(1, NWORKERS):
        g = jnp.maximum(g, wmv[pl.ds(i * 16, 16)])
    lanes = lax.iota(jnp.int32, 16)
    for s in (1, 2, 4, 8):
        tmp[...] = g
        g = jnp.maximum(g, plsc.load_gather(tmp, [lanes ^ s]))
    return g


TROW = NPAD * H // 128
TR_SLICE = TROW // NSUB


@functools.partial(
    pl.kernel,
    out_type=jax.ShapeDtypeStruct((NCORES * TROW, 128), jnp.float32),
    mesh=_sc_mesh,
    compiler_params=pltpu.CompilerParams(needs_layout_passes=False),
    scratch_types=[
        pltpu.VMEM((CHUNK,), jnp.int32),
        pltpu.VMEM((CHUNK, 16), jnp.float32),
        pltpu.VMEM((NWORKERS * 16,), jnp.float32),
        pltpu.VMEM((16,), jnp.float32),
        pltpu.VMEM((TROW,), jnp.int32),
        pltpu.VMEM((TROW, 128), jnp.float32),
        pltpu.VMEM_SHARED((TROW, 128), jnp.float32),
    ],
)
def _sc_denom(lg_hbm, d_hbm, wmax_hbm, dpart_hbm,
              dv, lgb, wmv, tmp16, rowidx, ptab, dtab):
    c = lax.axis_index("c")
    sid = lax.axis_index("s")
    wid = c * NSUB + sid
    lanes = lax.iota(jnp.int32, 16)
    lmask = lanes < H
    pltpu.sync_copy(wmax_hbm, wmv)
    gmax = _gmax_mem(wmv, tmp16)

    def init_body(r, _):
        for j in range(8):
            ptab[r, pl.ds(j * 16, 16)] = jnp.zeros((16,), jnp.float32)
        return 0

    lax.fori_loop(0, TROW, init_body, 0)

    def ridx_body(g, _):
        rowidx[pl.ds(g * 16, 16)] = g * 16 + lanes
        return 0

    lax.fori_loop(0, TROW // 16, ridx_body, 0)
    pltpu.sync_copy(ptab.at[pl.ds(0, TR_SLICE)],
                    dtab.at[pl.ds(sid * TR_SLICE, TR_SLICE)])

    def chunk_body(t, _):
        base = (wid + NWORKERS * t) * CHUNK
        pltpu.sync_copy(d_hbm.at[pl.ds(base, CHUNK)], dv)
        pltpu.sync_copy(lg_hbm.at[pl.ds(base, CHUNK)], lgb)

        def edge_body(e, _2):
            ev_row = jnp.exp(lgb[e, :] - gmax)
            dvec = plsc.load_gather(dv, [jnp.full((16,), e, jnp.int32)])
            tgt = dvec * H + lanes
            plsc.addupdate_scatter(
                ptab, [lax.shift_right_logical(tgt, 7), tgt & 127],
                ev_row, mask=lmask)
            return 0

        lax.fori_loop(0, CHUNK, edge_body, 0)
        return 0

    lax.fori_loop(0, _ntiles_chunks(wid), chunk_body, 0)
    plsc.subcore_barrier()
    pltpu.sync_copy(ptab, dtab.at[rowidx], add=True)
    plsc.subcore_barrier()
    pltpu.sync_copy(dtab.at[pl.ds(sid * TR_SLICE, TR_SLICE)],
                    dpart_hbm.at[pl.ds(c * TROW + sid * TR_SLICE, TR_SLICE)])


@functools.partial(
    pl.kernel,
    out_type=jax.ShapeDtypeStruct((E, 16), jnp.float32),
    mesh=_sc_mesh,
    compiler_params=pltpu.CompilerParams(needs_layout_passes=False),
    scratch_types=[
        pltpu.VMEM((CHUNK,), jnp.int32),
        pltpu.VMEM((CHUNK, 16), jnp.float32),
        pltpu.VMEM((CHUNK, 16), jnp.float32),
        pltpu.VMEM((NWORKERS * 16,), jnp.float32),
        pltpu.VMEM((16,), jnp.float32),
        pltpu.VMEM((TROW, 128), jnp.float32),
        pltpu.VMEM((TR_SLICE, 128), jnp.float32),
    ],
)
def _sc_coef(lg_hbm, d_hbm, wmax_hbm, dpart_hbm, coef_hbm,
             dv, lgb, cfb, wmv, tmp16, ptab, buf):
    wid = _worker_id()
    lanes = lax.iota(jnp.int32, 16)
    lmask = lanes < H
    pltpu.sync_copy(wmax_hbm, wmv)
    gmax = _gmax_mem(wmv, tmp16)
    pltpu.sync_copy(dpart_hbm.at[pl.ds(0, TROW)], ptab)
    for blk in range(NSUB):
        pltpu.sync_copy(
            dpart_hbm.at[pl.ds(TROW + blk * TR_SLICE, TR_SLICE)], buf)

        def add_body(r, _):
            for j in range(8):
                ptab[blk * TR_SLICE + r, pl.ds(j * 16, 16)] = (
                    ptab[blk * TR_SLICE + r, pl.ds(j * 16, 16)]
                    + buf[r, pl.ds(j * 16, 16)])
            return 0

        lax.fori_loop(0, TR_SLICE, add_body, 0)

    def chunk_body(t, _):
        base = (wid + NWORKERS * t) * CHUNK
        pltpu.sync_copy(d_hbm.at[pl.ds(base, CHUNK)], dv)
        pltpu.sync_copy(lg_hbm.at[pl.ds(base, CHUNK)], lgb)

        def edge_body(e, _2):
            ev_row = jnp.exp(lgb[e, :] - gmax)
            dvec = plsc.load_gather(dv, [jnp.full((16,), e, jnp.int32)])
            tgt = jnp.where(lmask, dvec * H + lanes, 0)
            den = plsc.load_gather(
                ptab, [lax.shift_right_logical(tgt, 7), tgt & 127])
            cfb[e, :] = ev_row / (den + 1e-16)
            return 0

        lax.fori_loop(0, CHUNK, edge_body, 0)
        pltpu.sync_copy(cfb, coef_hbm.at[pl.ds(base, CHUNK)])
        return 0

    lax.fori_loop(0, _ntiles_chunks(wid), chunk_body, 0)


@functools.partial(
    pl.kernel,
    out_type=jax.ShapeDtypeStruct((NCORES * 4 * NPAD, 128), jnp.float32),
    mesh=_sc_mesh,
    compiler_params=pltpu.CompilerParams(needs_layout_passes=False),
    scratch_types=[
        pltpu.VMEM((CHUNK,), jnp.int32),
        pltpu.VMEM((CHUNK,), jnp.int32),
        pltpu.VMEM((CHUNK,), jnp.int32),
        pltpu.VMEM((CHUNK,), jnp.int32),
        pltpu.VMEM((CHUNK, 16), jnp.float32),
        pltpu.VMEM((CHUNK, 128), jnp.float32),
        pltpu.VMEM((CHUNK, 128), jnp.float32),
        pltpu.VMEM((CHUNK, 128), jnp.float32),
        pltpu.VMEM_SHARED((NPAD, 128), jnp.float32),
        pltpu.SemaphoreType.DMA,
        pltpu.SemaphoreType.DMA,
    ],
)
def _sc_msg(vrel4_hbm, s_hbm, d_hbm, coef_hbm, opart_hbm,
            sv, dv, s4v0, s4v1, cfb, vr0, vr1, msgb, otab, sm0, sm1):
    c = lax.axis_index("c")
    sid = lax.axis_index("s")
    wid = c * NSUB + sid
    s4s, vrs, sms = (s4v0, s4v1), (vr0, vr1), (sm0, sm1)
    MMAIN = (NCHUNKS // NWORKERS) & ~1
    MEXTRA = NCHUNKS - MMAIN * NWORKERS

    def zero_body(r, _):
        for j in range(8):
            msgb[r, pl.ds(j * 16, 16)] = jnp.zeros((16,), jnp.float32)
        return 0

    for p in range(4):
        lax.fori_loop(0, CHUNK, zero_body, 0)
        for z in range(ROWS_PER_TILE // CHUNK):
            pltpu.sync_copy(
                msgb, otab.at[pl.ds(sid * ROWS_PER_TILE + z * CHUNK, CHUNK)])
        plsc.subcore_barrier()

        def issue(t, b):
            base = (wid + NWORKERS * t) * CHUNK
            pltpu.sync_copy(s_hbm.at[pl.ds(base, CHUNK)], sv)
            for g in range(CHUNK // 16):
                s4s[b][pl.ds(g * 16, 16)] = sv[pl.ds(g * 16, 16)] * 4 + p
            pltpu.async_copy(vrel4_hbm.at[s4s[b]], vrs[b], sms[b])

        def compute(t, b):
            base = (wid + NWORKERS * t) * CHUNK
            pltpu.sync_copy(d_hbm.at[pl.ds(base, CHUNK)], dv)
            pltpu.sync_copy(coef_hbm.at[pl.ds(base, CHUNK)], cfb)
            pltpu.make_async_copy(vrel4_hbm.at[s4s[b]], vrs[b], sms[b]).wait()
            vrows = vrs[b]

            def edge_body(e2, _2):
                for u in range(2):
                    e = e2 * 2 + u
                    crow = cfb[e, :]
                    c0 = crow[2 * p]
                    c1 = crow[2 * p + 1]
                    for j in range(8):
                        cc = c0 if j < 4 else c1
                        msgb[e, pl.ds(j * 16, 16)] = (
                            vrows[e, pl.ds(j * 16, 16)] * cc)
                return 0

            lax.fori_loop(0, CHUNK // 2, edge_body, 0)
            pltpu.sync_copy(msgb, otab.at[dv], add=True)

        issue(0, 0)
        issue(1, 1)

        def pair_body(t2, _):
            for b in range(2):
                t = t2 * 2 + b
                compute(t, b)

                @pl.when(t + 2 < MMAIN)
                def _():
                    issue(t + 2, b)
            return 0

        lax.fori_loop(0, MMAIN // 2, pair_body, 0)

        @pl.when(wid < MEXTRA)
        def _():
            t_tail = MMAIN * NWORKERS + wid
            base = t_tail * CHUNK
            pltpu.sync_copy(s_hbm.at[pl.ds(base, CHUNK)], sv)
            for g in range(CHUNK // 16):
                s4v0[pl.ds(g * 16, 16)] = sv[pl.ds(g * 16, 16)] * 4 + p
            pltpu.sync_copy(d_hbm.at[pl.ds(base, CHUNK)], dv)
            pltpu.sync_copy(coef_hbm.at[pl.ds(base, CHUNK)], cfb)
            pltpu.async_copy(vrel4_hbm.at[s4v0], vr0, sm0).wait()

            def edge_body(e2, _2):
                for u in range(2):
                    e = e2 * 2 + u
                    crow = cfb[e, :]
                    c0 = crow[2 * p]
                    c1 = crow[2 * p + 1]
                    for j in range(8):
                        cc = c0 if j < 4 else c1
                        msgb[e, pl.ds(j * 16, 16)] = (
                            vr0[e, pl.ds(j * 16, 16)] * cc)
                return 0

            lax.fori_loop(0, CHUNK // 2, edge_body, 0)
            pltpu.sync_copy(msgb, otab.at[dv], add=True)

        plsc.subcore_barrier()
        pltpu.sync_copy(
            otab.at[pl.ds(sid * ROWS_PER_TILE, ROWS_PER_TILE)],
            opart_hbm.at[pl.ds((c * 4 + p) * NPAD + sid * ROWS_PER_TILE,
                               ROWS_PER_TILE)])
        plsc.subcore_barrier()


def _edge_stage(q_dst, k_rel, v_rel, s_idx, d_idx, p_rel, n_dst):
    pvec = jnp.zeros((16,), jnp.float32).at[:H].set(p_rel / np.sqrt(D))
    q_p = lax.bitcast_convert_type(
        q_dst.astype(jnp.bfloat16).reshape(N, HID // 2, 2), jnp.int32)
    k_p = lax.bitcast_convert_type(
        k_rel.astype(jnp.bfloat16).reshape(N, HID // 2, 2), jnp.int32)
    lg, wmax = _sc_logits(q_p, k_p, s_idx, d_idx, pvec)
    dpart = _sc_denom(lg, d_idx, wmax)
    coef = _sc_coef(lg, d_idx, wmax, dpart)
    vrel4 = v_rel.reshape(N * 4, 128)
    opart = _sc_msg(vrel4, s_idx, d_idx, coef)
    o = opart.reshape(NCORES, 4, NPAD, 128).sum(0)
    return o.transpose(1, 0, 2).reshape(NPAD, HID)[:N]


def kernel(x_author, x_paper, edge_index__author__writes__paper,
           edge_index__paper__rev_writes__author, params):
    ei = {_ekey(ET[0]): edge_index__author__writes__paper,
          _ekey(ET[1]): edge_index__paper__rev_writes__author}
    x = {'author': _matmul(x_author, params['in']['author']['w'],
                           params['in']['author']['b'], act_out='relu'),
         'paper': _matmul(x_paper, params['in']['paper']['w'],
                          params['in']['paper']['b'], act_out='relu')}
    for layer in params['layers']:
        q = {t: _matmul(x[t], layer['q'][t]['w'], layer['q'][t]['b']) for t in NT}
        krel = {}
        vrel = {}
        for et in ET:
            src, _, dst = et
            r = layer['rel'][_ekey(et)]
            wk, bk = _combine_rel(layer['k'][src]['w'], layer['k'][src]['b'], r['a_rel'])
            wv, bv = _combine_rel(layer['v'][src]['w'], layer['v'][src]['b'], r['m_rel'])
            krel[_ekey(et)] = _matmul(x[src], wk, bk)
            vrel[_ekey(et)] = _matmul(x[src], wv, bv)
        out = {t: jnp.zeros((N, HID), jnp.float32) for t in NT}
        for et in ET:
            src, _, dst = et
            ek = _ekey(et)
            e = ei[ek]
            agg = _edge_stage(q[dst], krel[ek], vrel[ek], e[0], e[1],
                              layer['rel'][ek]['p_rel'], N)
            out[dst] = out[dst] + agg
        newx = {}
        for t in NT:
            beta = jax.nn.sigmoid(layer['skip'][t])
            w = layer['a'][t]['w'] * beta
            b = layer['a'][t]['b'] * beta
            newx[t] = _matmul(out[t], w, b, res=(1.0 - beta) * x[t], act_in='gelu')
        x = newx
    return _matmul(x['author'], params['out']['w'], params['out']['b'])


# speedup vs baseline: 1.1766x; 1.1766x over previous
import functools

import jax
import jax.numpy as jnp
import numpy as np
from jax import lax
from jax.experimental import pallas as pl
from jax.experimental.pallas import tpu as pltpu
from jax.experimental.pallas import tpu_sc as plsc

NT = ('author', 'paper')
ET = (('author', 'writes', 'paper'), ('paper', 'rev_writes', 'author'))
N = 10000
HID = 512
H = 8
D = 64
E = 160000

NCORES = 2
NSUB = 16
NWORKERS = NCORES * NSUB
CHUNK = 64
NCHUNKS = E // CHUNK
EXTRA = NCHUNKS - (NCHUNKS // NWORKERS) * NWORKERS
NPAD = 10240
ROWS_PER_TILE = NPAD // NSUB
NEG = np.float32(-3.0e38)

_sc_mesh = plsc.VectorSubcoreMesh(core_axis_name="c", subcore_axis_name="s")


def _ekey(et):
    return '__'.join(et)


def _mm_body(x_ref, w_ref, b_ref, o_ref, *, act_in, act_out):
    x = x_ref[...]
    if act_in == 'gelu':
        x = jax.nn.gelu(x)
    y = jnp.dot(x, w_ref[...], preferred_element_type=jnp.float32) + b_ref[...]
    if act_out == 'relu':
        y = jnp.maximum(y, 0.0)
    o_ref[...] = y


def _mm_res_body(x_ref, w_ref, b_ref, r_ref, o_ref, *, act_in, act_out):
    x = x_ref[...]
    if act_in == 'gelu':
        x = jax.nn.gelu(x)
    y = jnp.dot(x, w_ref[...], preferred_element_type=jnp.float32) + b_ref[...]
    o_ref[...] = y + r_ref[...]


def _matmul(x, w, b, res=None, act_in=None, act_out=None, block_m=1000):
    m, k = x.shape
    k2, n = w.shape
    assert k == k2 and m % block_m == 0
    grid = (m // block_m,)
    b2 = jnp.broadcast_to(b.reshape(1, n), (8, n))
    in_specs = [
        pl.BlockSpec((block_m, k), lambda i: (i, 0)),
        pl.BlockSpec((k, n), lambda i: (0, 0)),
        pl.BlockSpec((8, n), lambda i: (0, 0)),
    ]
    args = [x, w, b2]
    if res is None:
        body = functools.partial(_mm_body, act_in=act_in, act_out=act_out)
    else:
        body = functools.partial(_mm_res_body, act_in=act_in, act_out=act_out)
        in_specs.append(pl.BlockSpec((block_m, n), lambda i: (i, 0)))
        args.append(res)
    return pl.pallas_call(
        lambda *refs: body(refs[0], refs[1], refs[2][0:1], *refs[3:]),
        grid=grid,
        in_specs=in_specs,
        out_specs=pl.BlockSpec((block_m, n), lambda i: (i, 0)),
        out_shape=jax.ShapeDtypeStruct((m, n), jnp.float32),
    )(*args)


def _combine_rel(w, bvec, rel):
    bd = jax.scipy.linalg.block_diag if False else None
    eye = jnp.eye(H, dtype=jnp.float32)
    blk = jnp.einsum('hij,hg->higj', rel, eye).reshape(HID, HID)
    stack = jnp.concatenate([w, bvec.reshape(1, -1),
                             jnp.zeros((7, HID), jnp.float32)], axis=0)
    out = _matmul(stack, blk, jnp.zeros((HID,), jnp.float32), block_m=520)
    return out[:HID], out[HID]


def _ntiles_chunks(wid):
    return jnp.where(wid < EXTRA, NCHUNKS // NWORKERS + 1, NCHUNKS // NWORKERS)


_GDN = lax.GatherDimensionNumbers(
    offset_dims=(), collapsed_slice_dims=(0,), start_index_map=(0,))


def _lane_perm(x, idx):
    return lax.gather(x, idx[:, None], _GDN, slice_sizes=(1,),
                      mode=lax.GatherScatterMode.PROMISE_IN_BOUNDS)


def _bf_sum(x):
    lanes = lax.iota(jnp.int32, 16)
    for s in (1, 2, 4, 8):
        x = x + _lane_perm(x, lanes ^ s)
    return x


def _bf_max(x):
    lanes = lax.iota(jnp.int32, 16)
    for s in (1, 2, 4, 8):
        x = jnp.maximum(x, _lane_perm(x, lanes ^ s))
    return x


def _worker_id():
    return lax.axis_index("c") * NSUB + lax.axis_index("s")


LCH = 32
LNCH = E // LCH
LMAIN = (LNCH // NWORKERS) & ~1
LEXTRA = LNCH - LMAIN * NWORKERS


@functools.partial(
    pl.kernel,
    out_type=[jax.ShapeDtypeStruct((E, 16), jnp.float32),
              jax.ShapeDtypeStruct((NWORKERS * 16,), jnp.float32)],
    mesh=_sc_mesh,
    compiler_params=pltpu.CompilerParams(needs_layout_passes=False),
    scratch_types=[
        pltpu.VMEM((LCH,), jnp.int32),
        pltpu.VMEM((LCH,), jnp.int32),
        pltpu.VMEM((LCH,), jnp.int32),
        pltpu.VMEM((LCH,), jnp.int32),
        pltpu.VMEM((LCH, HID), jnp.float32),
        pltpu.VMEM((LCH, HID), jnp.float32),
        pltpu.VMEM((LCH, HID), jnp.float32),
        pltpu.VMEM((LCH, HID), jnp.float32),
        pltpu.VMEM((LCH, 16), jnp.float32),
        pltpu.VMEM((16,), jnp.float32),
        pltpu.VMEM((16,), jnp.float32),
        pltpu.SemaphoreType.DMA,
        pltpu.SemaphoreType.DMA,
        pltpu.SemaphoreType.DMA,
        pltpu.SemaphoreType.DMA,
    ],
)
def _sc_logits(q_hbm, k_hbm, s_hbm, d_hbm, p_hbm, lg_hbm, wmax_hbm,
               sv0, sv1, dv0, dv1, k0, k1, q0, q1, lgv, pv, mxv,
               sk0, sk1, sq0, sq1):
    wid = _worker_id()
    pltpu.sync_copy(p_hbm, pv)
    pvv = pv[...]
    lanes = lax.iota(jnp.int32, 16)
    svs, dvs, ks, qs = (sv0, sv1), (dv0, dv1), (k0, k1), (q0, q1)
    sks, sqs = (sk0, sk1), (sq0, sq1)

    def issue(t, b):
        base = (wid + NWORKERS * t) * LCH
        pltpu.sync_copy(s_hbm.at[pl.ds(base, LCH)], svs[b])
        pltpu.sync_copy(d_hbm.at[pl.ds(base, LCH)], dvs[b])
        pltpu.async_copy(k_hbm.at[svs[b]], ks[b], sks[b])
        pltpu.async_copy(q_hbm.at[dvs[b]], qs[b], sqs[b])

    def compute(t, b, mx):
        base = (wid + NWORKERS * t) * LCH
        pltpu.make_async_copy(k_hbm.at[svs[b]], ks[b], sks[b]).wait()
        pltpu.make_async_copy(q_hbm.at[dvs[b]], qs[b], sqs[b]).wait()
        krows, qrows = ks[b], qs[b]

        def edge_body(e2, mx2):
            for u in range(4):
                e = e2 * 4 + u
                row = jnp.zeros((16,), jnp.float32)
                for h in range(H):
                    acc = (qrows[e, pl.ds(h * D, 16)]
                           * krows[e, pl.ds(h * D, 16)])
                    for j in range(1, D // 16):
                        acc = acc + (qrows[e, pl.ds(h * D + j * 16, 16)]
                                     * krows[e, pl.ds(h * D + j * 16, 16)])
                    row = jnp.where(lanes == h, _bf_sum(acc), row)
                row = row * pvv
                lgv[e, :] = row
                mx2 = jnp.maximum(mx2, row)
            return mx2

        mx = lax.fori_loop(0, LCH // 4, edge_body, mx)
        pltpu.sync_copy(lgv, lg_hbm.at[pl.ds(base, LCH)])
        return mx

    issue(0, 0)
    issue(1, 1)

    def pair_body(t2, mx):
        for b in range(2):
            t = t2 * 2 + b
            mx = compute(t, b, mx)

            @pl.when(t + 2 < LMAIN)
            def _():
                issue(t + 2, b)
        return mx

    mx = lax.fori_loop(0, LMAIN // 2, pair_body,
                       jnp.zeros((16,), jnp.float32))

    mxv[...] = mx

    @pl.when(wid < LEXTRA)
    def _():
        base = (LMAIN * NWORKERS + wid) * LCH
        pltpu.sync_copy(s_hbm.at[pl.ds(base, LCH)], sv0)
        pltpu.sync_copy(d_hbm.at[pl.ds(base, LCH)], dv0)
        pltpu.async_copy(k_hbm.at[sv0], k0, sk0).wait()
        pltpu.async_copy(q_hbm.at[dv0], q0, sq0).wait()

        def edge_body(e2, _2):
            for u in range(2):
                e = e2 * 2 + u
                row = jnp.zeros((16,), jnp.float32)
                for h in range(H):
                    acc = (q0[e, pl.ds(h * D, 16)] * k0[e, pl.ds(h * D, 16)])
                    for j in range(1, D // 16):
                        acc = acc + (q0[e, pl.ds(h * D + j * 16, 16)]
                                     * k0[e, pl.ds(h * D + j * 16, 16)])
                    row = jnp.where(lanes == h, _bf_sum(acc), row)
                row = row * pvv
                lgv[e, :] = row
                mxv[...] = jnp.maximum(mxv[...], row)
            return 0

        lax.fori_loop(0, LCH // 2, edge_body, 0)
        pltpu.sync_copy(lgv, lg_hbm.at[pl.ds(base, LCH)])

    mxv[...] = _bf_max(mxv[...])
    pltpu.sync_copy(mxv, wmax_hbm.at[pl.ds(wid * 16, 16)])


def _gmax_of(wmv):
    g = wmv[pl.ds(0, 16)]
    for i in range(1, NWORKERS):
        g = jnp.maximum(g, wmv[pl.ds(i * 16, 16)])
    return _bf_max(g)


def _gmax_mem(wmv, tmp):
    g = wmv[pl.ds(0, 16)]
    for i in range(1, NWORKERS):
        g = jnp.maximum(g, wmv[pl.ds(i * 16, 16)])
    lanes = lax.iota(jnp.int32, 16)
    for s in (1, 2, 4, 8):
        tmp[...] = g
        g = jnp.maximum(g, plsc.load_gather(tmp, [lanes ^ s]))
    return g


TROW = NPAD * H // 128
TR_SLICE = TROW // NSUB


@functools.partial(
    pl.kernel,
    out_type=jax.ShapeDtypeStruct((NCORES * TROW, 128), jnp.float32),
    mesh=_sc_mesh,
    compiler_params=pltpu.CompilerParams(needs_layout_passes=False),
    scratch_types=[
        pltpu.VMEM((CHUNK,), jnp.int32),
        pltpu.VMEM((CHUNK, 16), jnp.float32),
        pltpu.VMEM((NWORKERS * 16,), jnp.float32),
        pltpu.VMEM((16,), jnp.float32),
        pltpu.VMEM((TROW,), jnp.int32),
        pltpu.VMEM((TROW, 128), jnp.float32),
        pltpu.VMEM_SHARED((TROW, 128), jnp.float32),
    ],
)
def _sc_denom(lg_hbm, d_hbm, wmax_hbm, dpart_hbm,
              dv, lgb, wmv, tmp16, rowidx, ptab, dtab):
    c = lax.axis_index("c")
    sid = lax.axis_index("s")
    wid = c * NSUB + sid
    lanes = lax.iota(jnp.int32, 16)
    lmask = lanes < H
    pltpu.sync_copy(wmax_hbm, wmv)
    gmax = _gmax_mem(wmv, tmp16)

    def init_body(r, _):
        for j in range(8):
            ptab[r, pl.ds(j * 16, 16)] = jnp.zeros((16,), jnp.float32)
        return 0

    lax.fori_loop(0, TROW, init_body, 0)

    def ridx_body(g, _):
        rowidx[pl.ds(g * 16, 16)] = g * 16 + lanes
        return 0

    lax.fori_loop(0, TROW // 16, ridx_body, 0)
    pltpu.sync_copy(ptab.at[pl.ds(0, TR_SLICE)],
                    dtab.at[pl.ds(sid * TR_SLICE, TR_SLICE)])

    def chunk_body(t, _):
        base = (wid + NWORKERS * t) * CHUNK
        pltpu.sync_copy(d_hbm.at[pl.ds(base, CHUNK)], dv)
        pltpu.sync_copy(lg_hbm.at[pl.ds(base, CHUNK)], lgb)

        def edge_body(e, _2):
            ev_row = jnp.exp(lgb[e, :] - gmax)
            dvec = plsc.load_gather(dv, [jnp.full((16,), e, jnp.int32)])
            tgt = dvec * H + lanes
            plsc.addupdate_scatter(
                ptab, [lax.shift_right_logical(tgt, 7), tgt & 127],
                ev_row, mask=lmask)
            return 0

        lax.fori_loop(0, CHUNK, edge_body, 0)
        return 0

    lax.fori_loop(0, _ntiles_chunks(wid), chunk_body, 0)
    plsc.subcore_barrier()
    pltpu.sync_copy(ptab, dtab.at[rowidx], add=True)
    plsc.subcore_barrier()
    pltpu.sync_copy(dtab.at[pl.ds(sid * TR_SLICE, TR_SLICE)],
                    dpart_hbm.at[pl.ds(c * TROW + sid * TR_SLICE, TR_SLICE)])


@functools.partial(
    pl.kernel,
    out_type=jax.ShapeDtypeStruct((E, 16), jnp.float32),
    mesh=_sc_mesh,
    compiler_params=pltpu.CompilerParams(needs_layout_passes=False),
    scratch_types=[
        pltpu.VMEM((CHUNK,), jnp.int32),
        pltpu.VMEM((CHUNK, 16), jnp.float32),
        pltpu.VMEM((CHUNK, 16), jnp.float32),
        pltpu.VMEM((NWORKERS * 16,), jnp.float32),
        pltpu.VMEM((16,), jnp.float32),
        pltpu.VMEM((TROW, 128), jnp.float32),
        pltpu.VMEM((TR_SLICE, 128), jnp.float32),
    ],
)
def _sc_coef(lg_hbm, d_hbm, wmax_hbm, dpart_hbm, coef_hbm,
             dv, lgb, cfb, wmv, tmp16, ptab, buf):
    wid = _worker_id()
    lanes = lax.iota(jnp.int32, 16)
    lmask = lanes < H
    pltpu.sync_copy(wmax_hbm, wmv)
    gmax = _gmax_mem(wmv, tmp16)
    pltpu.sync_copy(dpart_hbm.at[pl.ds(0, TROW)], ptab)
    for blk in range(NSUB):
        pltpu.sync_copy(
            dpart_hbm.at[pl.ds(TROW + blk * TR_SLICE, TR_SLICE)], buf)

        def add_body(r, _):
            for j in range(8):
                ptab[blk * TR_SLICE + r, pl.ds(j * 16, 16)] = (
                    ptab[blk * TR_SLICE + r, pl.ds(j * 16, 16)]
                    + buf[r, pl.ds(j * 16, 16)])
            return 0

        lax.fori_loop(0, TR_SLICE, add_body, 0)

    def chunk_body(t, _):
        base = (wid + NWORKERS * t) * CHUNK
        pltpu.sync_copy(d_hbm.at[pl.ds(base, CHUNK)], dv)
        pltpu.sync_copy(lg_hbm.at[pl.ds(base, CHUNK)], lgb)

        def edge_body(e, _2):
            ev_row = jnp.exp(lgb[e, :] - gmax)
            dvec = plsc.load_gather(dv, [jnp.full((16,), e, jnp.int32)])
            tgt = jnp.where(lmask, dvec * H + lanes, 0)
            den = plsc.load_gather(
                ptab, [lax.shift_right_logical(tgt, 7), tgt & 127])
            cfb[e, :] = ev_row / (den + 1e-16)
            return 0

        lax.fori_loop(0, CHUNK, edge_body, 0)
        pltpu.sync_copy(cfb, coef_hbm.at[pl.ds(base, CHUNK)])
        return 0

    lax.fori_loop(0, _ntiles_chunks(wid), chunk_body, 0)


@functools.partial(
    pl.kernel,
    out_type=jax.ShapeDtypeStruct((NCORES * 4 * NPAD, 128), jnp.float32),
    mesh=_sc_mesh,
    compiler_params=pltpu.CompilerParams(needs_layout_passes=False),
    scratch_types=[
        pltpu.VMEM((CHUNK,), jnp.int32),
        pltpu.VMEM((CHUNK,), jnp.int32),
        pltpu.VMEM((CHUNK,), jnp.int32),
        pltpu.VMEM((CHUNK,), jnp.int32),
        pltpu.VMEM((CHUNK, 16), jnp.float32),
        pltpu.VMEM((CHUNK, 128), jnp.float32),
        pltpu.VMEM((CHUNK, 128), jnp.float32),
        pltpu.VMEM((CHUNK, 128), jnp.float32),
        pltpu.VMEM_SHARED((NPAD, 128), jnp.float32),
        pltpu.SemaphoreType.DMA,
        pltpu.SemaphoreType.DMA,
    ],
)
def _sc_msg(vrel4_hbm, s_hbm, d_hbm, coef_hbm, opart_hbm,
            sv, dv, s4v0, s4v1, cfb, vr0, vr1, msgb, otab, sm0, sm1):
    c = lax.axis_index("c")
    sid = lax.axis_index("s")
    wid = c * NSUB + sid
    s4s, vrs, sms = (s4v0, s4v1), (vr0, vr1), (sm0, sm1)
    MMAIN = (NCHUNKS // NWORKERS) & ~1
    MEXTRA = NCHUNKS - MMAIN * NWORKERS

    def zero_body(r, _):
        for j in range(8):
            msgb[r, pl.ds(j * 16, 16)] = jnp.zeros((16,), jnp.float32)
        return 0

    for p in range(4):
        lax.fori_loop(0, CHUNK, zero_body, 0)
        for z in range(ROWS_PER_TILE // CHUNK):
            pltpu.sync_copy(
                msgb, otab.at[pl.ds(sid * ROWS_PER_TILE + z * CHUNK, CHUNK)])
        plsc.subcore_barrier()

        def issue(t, b):
            base = (wid + NWORKERS * t) * CHUNK
            pltpu.sync_copy(s_hbm.at[pl.ds(base, CHUNK)], sv)
            for g in range(CHUNK // 16):
                s4s[b][pl.ds(g * 16, 16)] = sv[pl.ds(g * 16, 16)] * 4 + p
            pltpu.async_copy(vrel4_hbm.at[s4s[b]], vrs[b], sms[b])

        def compute(t, b):
            base = (wid + NWORKERS * t) * CHUNK
            pltpu.sync_copy(d_hbm.at[pl.ds(base, CHUNK)], dv)
            pltpu.sync_copy(coef_hbm.at[pl.ds(base, CHUNK)], cfb)
            pltpu.make_async_copy(vrel4_hbm.at[s4s[b]], vrs[b], sms[b]).wait()
            vrows = vrs[b]

            def edge_body(e2, _2):
                for u in range(2):
                    e = e2 * 2 + u
                    crow = cfb[e, :]
                    c0 = crow[2 * p]
                    c1 = crow[2 * p + 1]
                    for j in range(8):
                        cc = c0 if j < 4 else c1
                        msgb[e, pl.ds(j * 16, 16)] = (
                            vrows[e, pl.ds(j * 16, 16)] * cc)
                return 0

            lax.fori_loop(0, CHUNK // 2, edge_body, 0)
            pltpu.sync_copy(msgb, otab.at[dv], add=True)

        issue(0, 0)
        issue(1, 1)

        def pair_body(t2, _):
            for b in range(2):
                t = t2 * 2 + b
                compute(t, b)

                @pl.when(t + 2 < MMAIN)
                def _():
                    issue(t + 2, b)
            return 0

        lax.fori_loop(0, MMAIN // 2, pair_body, 0)

        @pl.when(wid < MEXTRA)
        def _():
            t_tail = MMAIN * NWORKERS + wid
            base = t_tail * CHUNK
            pltpu.sync_copy(s_hbm.at[pl.ds(base, CHUNK)], sv)
            for g in range(CHUNK // 16):
                s4v0[pl.ds(g * 16, 16)] = sv[pl.ds(g * 16, 16)] * 4 + p
            pltpu.sync_copy(d_hbm.at[pl.ds(base, CHUNK)], dv)
            pltpu.sync_copy(coef_hbm.at[pl.ds(base, CHUNK)], cfb)
            pltpu.async_copy(vrel4_hbm.at[s4v0], vr0, sm0).wait()

            def edge_body(e2, _2):
                for u in range(2):
                    e = e2 * 2 + u
                    crow = cfb[e, :]
                    c0 = crow[2 * p]
                    c1 = crow[2 * p + 1]
                    for j in range(8):
                        cc = c0 if j < 4 else c1
                        msgb[e, pl.ds(j * 16, 16)] = (
                            vr0[e, pl.ds(j * 16, 16)] * cc)
                return 0

            lax.fori_loop(0, CHUNK // 2, edge_body, 0)
            pltpu.sync_copy(msgb, otab.at[dv], add=True)

        plsc.subcore_barrier()
        pltpu.sync_copy(
            otab.at[pl.ds(sid * ROWS_PER_TILE, ROWS_PER_TILE)],
            opart_hbm.at[pl.ds((c * 4 + p) * NPAD + sid * ROWS_PER_TILE,
                               ROWS_PER_TILE)])
        plsc.subcore_barrier()


def _edge_stage(q_dst, k_rel, v_rel, s_idx, d_idx, p_rel, n_dst):
    pvec = jnp.zeros((16,), jnp.float32).at[:H].set(p_rel / np.sqrt(D))
    lg, wmax = _sc_logits(q_dst, k_rel, s_idx, d_idx, pvec)
    dpart = _sc_denom(lg, d_idx, wmax)
    coef = _sc_coef(lg, d_idx, wmax, dpart)
    vrel4 = v_rel.reshape(N * 4, 128)
    opart = _sc_msg(vrel4, s_idx, d_idx, coef)
    o = opart.reshape(NCORES, 4, NPAD, 128).sum(0)
    return o.transpose(1, 0, 2).reshape(NPAD, HID)[:N]


def kernel(x_author, x_paper, edge_index__author__writes__paper,
           edge_index__paper__rev_writes__author, params):
    ei = {_ekey(ET[0]): edge_index__author__writes__paper,
          _ekey(ET[1]): edge_index__paper__rev_writes__author}
    x = {'author': _matmul(x_author, params['in']['author']['w'],
                           params['in']['author']['b'], act_out='relu'),
         'paper': _matmul(x_paper, params['in']['paper']['w'],
                          params['in']['paper']['b'], act_out='relu')}
    for layer in params['layers']:
        q = {t: _matmul(x[t], layer['q'][t]['w'], layer['q'][t]['b']) for t in NT}
        krel = {}
        vrel = {}
        for et in ET:
            src, _, dst = et
            r = layer['rel'][_ekey(et)]
            wk, bk = _combine_rel(layer['k'][src]['w'], layer['k'][src]['b'], r['a_rel'])
            wv, bv = _combine_rel(layer['v'][src]['w'], layer['v'][src]['b'], r['m_rel'])
            krel[_ekey(et)] = _matmul(x[src], wk, bk)
            vrel[_ekey(et)] = _matmul(x[src], wv, bv)
        out = {t: jnp.zeros((N, HID), jnp.float32) for t in NT}
        for et in ET:
            src, _, dst = et
            ek = _ekey(et)
            e = ei[ek]
            agg = _edge_stage(q[dst], krel[ek], vrel[ek], e[0], e[1],
                              layer['rel'][ek]['p_rel'], N)
            out[dst] = out[dst] + agg
        newx = {}
        for t in NT:
            beta = jax.nn.sigmoid(layer['skip'][t])
            w = layer['a'][t]['w'] * beta
            b = layer['a'][t]['b'] * beta
            newx[t] = _matmul(out[t], w, b, res=(1.0 - beta) * x[t], act_in='gelu')
        x = newx
    return _matmul(x['author'], params['out']['w'], params['out']['b'])


# speedup vs baseline: 1.1935x; 1.0143x over previous
import functools

import jax
import jax.numpy as jnp
import numpy as np
from jax import lax
from jax.experimental import pallas as pl
from jax.experimental.pallas import tpu as pltpu
from jax.experimental.pallas import tpu_sc as plsc

NT = ('author', 'paper')
ET = (('author', 'writes', 'paper'), ('paper', 'rev_writes', 'author'))
N = 10000
HID = 512
H = 8
D = 64
E = 160000

NCORES = 2
NSUB = 16
NWORKERS = NCORES * NSUB
CHUNK = 64
NCHUNKS = E // CHUNK
EXTRA = NCHUNKS - (NCHUNKS // NWORKERS) * NWORKERS
NPAD = 10240
ROWS_PER_TILE = NPAD // NSUB
NEG = np.float32(-3.0e38)

_sc_mesh = plsc.VectorSubcoreMesh(core_axis_name="c", subcore_axis_name="s")


def _ekey(et):
    return '__'.join(et)


def _mm_body(x_ref, w_ref, b_ref, o_ref, *, act_in, act_out):
    x = x_ref[...]
    if act_in == 'gelu':
        x = jax.nn.gelu(x)
    y = jnp.dot(x, w_ref[...], preferred_element_type=jnp.float32) + b_ref[...]
    if act_out == 'relu':
        y = jnp.maximum(y, 0.0)
    o_ref[...] = y


def _mm_res_body(x_ref, w_ref, b_ref, r_ref, o_ref, *, act_in, act_out):
    x = x_ref[...]
    if act_in == 'gelu':
        x = jax.nn.gelu(x)
    y = jnp.dot(x, w_ref[...], preferred_element_type=jnp.float32) + b_ref[...]
    o_ref[...] = y + r_ref[...]


def _matmul(x, w, b, res=None, act_in=None, act_out=None, block_m=1024):
    m, k = x.shape
    k2, n = w.shape
    assert k == k2 and m % block_m == 0
    grid = (m // block_m,)
    b2 = jnp.broadcast_to(b.reshape(1, n), (8, n))
    in_specs = [
        pl.BlockSpec((block_m, k), lambda i: (i, 0)),
        pl.BlockSpec((k, n), lambda i: (0, 0)),
        pl.BlockSpec((8, n), lambda i: (0, 0)),
    ]
    args = [x, w, b2]
    if res is None:
        body = functools.partial(_mm_body, act_in=act_in, act_out=act_out)
    else:
        body = functools.partial(_mm_res_body, act_in=act_in, act_out=act_out)
        in_specs.append(pl.BlockSpec((block_m, n), lambda i: (i, 0)))
        args.append(res)
    return pl.pallas_call(
        lambda *refs: body(refs[0], refs[1], refs[2][0:1], *refs[3:]),
        grid=grid,
        in_specs=in_specs,
        out_specs=pl.BlockSpec((block_m, n), lambda i: (i, 0)),
        out_shape=jax.ShapeDtypeStruct((m, n), jnp.float32),
    )(*args)


def _combine_rel(w, bvec, rel):
    bd = jax.scipy.linalg.block_diag if False else None
    eye = jnp.eye(H, dtype=jnp.float32)
    blk = jnp.einsum('hij,hg->higj', rel, eye).reshape(HID, HID)
    stack = jnp.concatenate([w, bvec.reshape(1, -1),
                             jnp.zeros((7, HID), jnp.float32)], axis=0)
    out = _matmul(stack, blk, jnp.zeros((HID,), jnp.float32), block_m=520)
    return out[:HID], out[HID]


def _agg_body(o_ref, w_ref, b_ref, r_ref, out_ref):
    acc = b_ref[0:1] + r_ref[...]
    o = o_ref[...]
    for pp in range(4):
        y = jax.nn.gelu(o[0, pp] + o[1, pp])
        acc = acc + jnp.dot(y, w_ref[...][pp],
                            preferred_element_type=jnp.float32)
    out_ref[...] = acc


def _matmul_agg(opart, w, b, res, block_m=1024):
    o4 = opart.reshape(NCORES, 4, NPAD, 128)
    w4 = w.reshape(4, 128, HID)
    b2 = jnp.broadcast_to(b.reshape(1, HID), (8, HID))
    grid = (NPAD // block_m,)
    return pl.pallas_call(
        _agg_body,
        grid=grid,
        in_specs=[
            pl.BlockSpec((NCORES, 4, block_m, 128), lambda i: (0, 0, i, 0)),
            pl.BlockSpec((4, 128, HID), lambda i: (0, 0, 0)),
            pl.BlockSpec((8, HID), lambda i: (0, 0)),
            pl.BlockSpec((block_m, HID), lambda i: (i, 0)),
        ],
        out_specs=pl.BlockSpec((block_m, HID), lambda i: (i, 0)),
        out_shape=jax.ShapeDtypeStruct((NPAD, HID), jnp.float32),
    )(o4, w4, b2, res)


def _ntiles_chunks(wid):
    return jnp.where(wid < EXTRA, NCHUNKS // NWORKERS + 1, NCHUNKS // NWORKERS)


_GDN = lax.GatherDimensionNumbers(
    offset_dims=(), collapsed_slice_dims=(0,), start_index_map=(0,))


def _lane_perm(x, idx):
    return lax.gather(x, idx[:, None], _GDN, slice_sizes=(1,),
                      mode=lax.GatherScatterMode.PROMISE_IN_BOUNDS)


def _bf_sum(x):
    lanes = lax.iota(jnp.int32, 16)
    for s in (1, 2, 4, 8):
        x = x + _lane_perm(x, lanes ^ s)
    return x


def _bf_max(x):
    lanes = lax.iota(jnp.int32, 16)
    for s in (1, 2, 4, 8):
        x = jnp.maximum(x, _lane_perm(x, lanes ^ s))
    return x


def _worker_id():
    return lax.axis_index("c") * NSUB + lax.axis_index("s")


LCH = 32
LNCH = E // LCH
LMAIN = (LNCH // NWORKERS) & ~1
LEXTRA = LNCH - LMAIN * NWORKERS


@functools.partial(
    pl.kernel,
    out_type=[jax.ShapeDtypeStruct((E, 16), jnp.float32),
              jax.ShapeDtypeStruct((NWORKERS * 16,), jnp.float32)],
    mesh=_sc_mesh,
    compiler_params=pltpu.CompilerParams(needs_layout_passes=False),
    scratch_types=[
        pltpu.VMEM((LCH,), jnp.int32),
        pltpu.VMEM((LCH,), jnp.int32),
        pltpu.VMEM((LCH,), jnp.int32),
        pltpu.VMEM((LCH,), jnp.int32),
        pltpu.VMEM((LCH, HID), jnp.float32),
        pltpu.VMEM((LCH, HID), jnp.float32),
        pltpu.VMEM((LCH, HID), jnp.float32),
        pltpu.VMEM((LCH, HID), jnp.float32),
        pltpu.VMEM((LCH, 16), jnp.float32),
        pltpu.VMEM((16,), jnp.float32),
        pltpu.VMEM((16,), jnp.float32),
        pltpu.SemaphoreType.DMA,
        pltpu.SemaphoreType.DMA,
        pltpu.SemaphoreType.DMA,
        pltpu.SemaphoreType.DMA,
    ],
)
def _sc_logits(q_hbm, k_hbm, s_hbm, d_hbm, p_hbm, lg_hbm, wmax_hbm,
               sv0, sv1, dv0, dv1, k0, k1, q0, q1, lgv, pv, mxv,
               sk0, sk1, sq0, sq1):
    wid = _worker_id()
    pltpu.sync_copy(p_hbm, pv)
    pvv = pv[...]
    lanes = lax.iota(jnp.int32, 16)
    svs, dvs, ks, qs = (sv0, sv1), (dv0, dv1), (k0, k1), (q0, q1)
    sks, sqs = (sk0, sk1), (sq0, sq1)

    def issue(t, b):
        base = (wid + NWORKERS * t) * LCH
        pltpu.sync_copy(s_hbm.at[pl.ds(base, LCH)], svs[b])
        pltpu.sync_copy(d_hbm.at[pl.ds(base, LCH)], dvs[b])
        pltpu.async_copy(k_hbm.at[svs[b]], ks[b], sks[b])
        pltpu.async_copy(q_hbm.at[dvs[b]], qs[b], sqs[b])

    def compute(t, b, mx):
        base = (wid + NWORKERS * t) * LCH
        pltpu.make_async_copy(k_hbm.at[svs[b]], ks[b], sks[b]).wait()
        pltpu.make_async_copy(q_hbm.at[dvs[b]], qs[b], sqs[b]).wait()
        krows, qrows = ks[b], qs[b]

        def edge_body(e2, mx2):
            for u in range(4):
                e = e2 * 4 + u
                row = jnp.zeros((16,), jnp.float32)
                for h in range(H):
                    acc = (qrows[e, pl.ds(h * D, 16)]
                           * krows[e, pl.ds(h * D, 16)])
                    for j in range(1, D // 16):
                        acc = acc + (qrows[e, pl.ds(h * D + j * 16, 16)]
                                     * krows[e, pl.ds(h * D + j * 16, 16)])
                    row = jnp.where(lanes == h, _bf_sum(acc), row)
                row = row * pvv
                lgv[e, :] = row
                mx2 = jnp.maximum(mx2, row)
            return mx2

        mx = lax.fori_loop(0, LCH // 4, edge_body, mx)
        pltpu.sync_copy(lgv, lg_hbm.at[pl.ds(base, LCH)])
        return mx

    issue(0, 0)
    issue(1, 1)

    def pair_body(t2, mx):
        for b in range(2):
            t = t2 * 2 + b
            mx = compute(t, b, mx)

            @pl.when(t + 2 < LMAIN)
            def _():
                issue(t + 2, b)
        return mx

    mx = lax.fori_loop(0, LMAIN // 2, pair_body,
                       jnp.zeros((16,), jnp.float32))

    mxv[...] = mx

    @pl.when(wid < LEXTRA)
    def _():
        base = (LMAIN * NWORKERS + wid) * LCH
        pltpu.sync_copy(s_hbm.at[pl.ds(base, LCH)], sv0)
        pltpu.sync_copy(d_hbm.at[pl.ds(base, LCH)], dv0)
        pltpu.async_copy(k_hbm.at[sv0], k0, sk0).wait()
        pltpu.async_copy(q_hbm.at[dv0], q0, sq0).wait()

        def edge_body(e2, _2):
            for u in range(2):
                e = e2 * 2 + u
                row = jnp.zeros((16,), jnp.float32)
                for h in range(H):
                    acc = (q0[e, pl.ds(h * D, 16)] * k0[e, pl.ds(h * D, 16)])
                    for j in range(1, D // 16):
                        acc = acc + (q0[e, pl.ds(h * D + j * 16, 16)]
                                     * k0[e, pl.ds(h * D + j * 16, 16)])
                    row = jnp.where(lanes == h, _bf_sum(acc), row)
                row = row * pvv
                lgv[e, :] = row
                mxv[...] = jnp.maximum(mxv[...], row)
            return 0

        lax.fori_loop(0, LCH // 2, edge_body, 0)
        pltpu.sync_copy(lgv, lg_hbm.at[pl.ds(base, LCH)])

    mxv[...] = _bf_max(mxv[...])
    pltpu.sync_copy(mxv, wmax_hbm.at[pl.ds(wid * 16, 16)])


def _gmax_of(wmv):
    g = wmv[pl.ds(0, 16)]
    for i in range(1, NWORKERS):
        g = jnp.maximum(g, wmv[pl.ds(i * 16, 16)])
    return _bf_max(g)


def _gmax_mem(wmv, tmp):
    g = wmv[pl.ds(0, 16)]
    for i in range(1, NWORKERS):
        g = jnp.maximum(g, wmv[pl.ds(i * 16, 16)])
    lanes = lax.iota(jnp.int32, 16)
    for s in (1, 2, 4, 8):
        tmp[...] = g
        g = jnp.maximum(g, plsc.load_gather(tmp, [lanes ^ s]))
    return g


TROW = NPAD * H // 128
TR_SLICE = TROW // NSUB


@functools.partial(
    pl.kernel,
    out_type=jax.ShapeDtypeStruct((NCORES * TROW, 128), jnp.float32),
    mesh=_sc_mesh,
    compiler_params=pltpu.CompilerParams(needs_layout_passes=False),
    scratch_types=[
        pltpu.VMEM((CHUNK,), jnp.int32),
        pltpu.VMEM((CHUNK, 16), jnp.float32),
        pltpu.VMEM((NWORKERS * 16,), jnp.float32),
        pltpu.VMEM((16,), jnp.float32),
        pltpu.VMEM((TROW,), jnp.int32),
        pltpu.VMEM((TROW, 128), jnp.float32),
        pltpu.VMEM_SHARED((TROW, 128), jnp.float32),
    ],
)
def _sc_denom(lg_hbm, d_hbm, wmax_hbm, dpart_hbm,
              dv, lgb, wmv, tmp16, rowidx, ptab, dtab):
    c = lax.axis_index("c")
    sid = lax.axis_index("s")
    wid = c * NSUB + sid
    lanes = lax.iota(jnp.int32, 16)
    lmask = lanes < H
    pltpu.sync_copy(wmax_hbm, wmv)
    gmax = _gmax_mem(wmv, tmp16)

    def init_body(r, _):
        for j in range(8):
            ptab[r, pl.ds(j * 16, 16)] = jnp.zeros((16,), jnp.float32)
        return 0

    lax.fori_loop(0, TROW, init_body, 0)

    def ridx_body(g, _):
        rowidx[pl.ds(g * 16, 16)] = g * 16 + lanes
        return 0

    lax.fori_loop(0, TROW // 16, ridx_body, 0)
    pltpu.sync_copy(ptab.at[pl.ds(0, TR_SLICE)],
                    dtab.at[pl.ds(sid * TR_SLICE, TR_SLICE)])

    def chunk_body(t, _):
        base = (wid + NWORKERS * t) * CHUNK
        pltpu.sync_copy(d_hbm.at[pl.ds(base, CHUNK)], dv)
        pltpu.sync_copy(lg_hbm.at[pl.ds(base, CHUNK)], lgb)

        def edge_body(e, _2):
            ev_row = jnp.exp(lgb[e, :] - gmax)
            dvec = plsc.load_gather(dv, [jnp.full((16,), e, jnp.int32)])
            tgt = dvec * H + lanes
            plsc.addupdate_scatter(
                ptab, [lax.shift_right_logical(tgt, 7), tgt & 127],
                ev_row, mask=lmask)
            return 0

        lax.fori_loop(0, CHUNK, edge_body, 0)
        return 0

    lax.fori_loop(0, _ntiles_chunks(wid), chunk_body, 0)
    plsc.subcore_barrier()
    pltpu.sync_copy(ptab, dtab.at[rowidx], add=True)
    plsc.subcore_barrier()
    pltpu.sync_copy(dtab.at[pl.ds(sid * TR_SLICE, TR_SLICE)],
                    dpart_hbm.at[pl.ds(c * TROW + sid * TR_SLICE, TR_SLICE)])


@functools.partial(
    pl.kernel,
    out_type=jax.ShapeDtypeStruct((E, 16), jnp.float32),
    mesh=_sc_mesh,
    compiler_params=pltpu.CompilerParams(needs_layout_passes=False),
    scratch_types=[
        pltpu.VMEM((CHUNK,), jnp.int32),
        pltpu.VMEM((CHUNK, 16), jnp.float32),
        pltpu.VMEM((CHUNK, 16), jnp.float32),
        pltpu.VMEM((NWORKERS * 16,), jnp.float32),
        pltpu.VMEM((16,), jnp.float32),
        pltpu.VMEM((TROW, 128), jnp.float32),
        pltpu.VMEM((TR_SLICE, 128), jnp.float32),
    ],
)
def _sc_coef(lg_hbm, d_hbm, wmax_hbm, dpart_hbm, coef_hbm,
             dv, lgb, cfb, wmv, tmp16, ptab, buf):
    wid = _worker_id()
    lanes = lax.iota(jnp.int32, 16)
    lmask = lanes < H
    pltpu.sync_copy(wmax_hbm, wmv)
    gmax = _gmax_mem(wmv, tmp16)
    pltpu.sync_copy(dpart_hbm.at[pl.ds(0, TROW)], ptab)
    for blk in range(NSUB):
        pltpu.sync_copy(
            dpart_hbm.at[pl.ds(TROW + blk * TR_SLICE, TR_SLICE)], buf)

        def add_body(r, _):
            for j in range(8):
                ptab[blk * TR_SLICE + r, pl.ds(j * 16, 16)] = (
                    ptab[blk * TR_SLICE + r, pl.ds(j * 16, 16)]
                    + buf[r, pl.ds(j * 16, 16)])
            return 0

        lax.fori_loop(0, TR_SLICE, add_body, 0)

    def chunk_body(t, _):
        base = (wid + NWORKERS * t) * CHUNK
        pltpu.sync_copy(d_hbm.at[pl.ds(base, CHUNK)], dv)
        pltpu.sync_copy(lg_hbm.at[pl.ds(base, CHUNK)], lgb)

        def edge_body(e, _2):
            ev_row = jnp.exp(lgb[e, :] - gmax)
            dvec = plsc.load_gather(dv, [jnp.full((16,), e, jnp.int32)])
            tgt = jnp.where(lmask, dvec * H + lanes, 0)
            den = plsc.load_gather(
                ptab, [lax.shift_right_logical(tgt, 7), tgt & 127])
            cfb[e, :] = ev_row / (den + 1e-16)
            return 0

        lax.fori_loop(0, CHUNK, edge_body, 0)
        pltpu.sync_copy(cfb, coef_hbm.at[pl.ds(base, CHUNK)])
        return 0

    lax.fori_loop(0, _ntiles_chunks(wid), chunk_body, 0)


@functools.partial(
    pl.kernel,
    out_type=jax.ShapeDtypeStruct((NCORES * 4 * NPAD, 128), jnp.float32),
    mesh=_sc_mesh,
    compiler_params=pltpu.CompilerParams(needs_layout_passes=False),
    scratch_types=[
        pltpu.VMEM((CHUNK,), jnp.int32),
        pltpu.VMEM((CHUNK,), jnp.int32),
        pltpu.VMEM((CHUNK,), jnp.int32),
        pltpu.VMEM((CHUNK,), jnp.int32),
        pltpu.VMEM((CHUNK, 16), jnp.float32),
        pltpu.VMEM((CHUNK, 128), jnp.float32),
        pltpu.VMEM((CHUNK, 128), jnp.float32),
        pltpu.VMEM((CHUNK, 128), jnp.float32),
        pltpu.VMEM_SHARED((NPAD, 128), jnp.float32),
        pltpu.SemaphoreType.DMA,
        pltpu.SemaphoreType.DMA,
    ],
)
def _sc_msg(vrel4_hbm, s_hbm, d_hbm, coef_hbm, opart_hbm,
            sv, dv, s4v0, s4v1, cfb, vr0, vr1, msgb, otab, sm0, sm1):
    c = lax.axis_index("c")
    sid = lax.axis_index("s")
    wid = c * NSUB + sid
    s4s, vrs, sms = (s4v0, s4v1), (vr0, vr1), (sm0, sm1)
    MMAIN = (NCHUNKS // NWORKERS) & ~1
    MEXTRA = NCHUNKS - MMAIN * NWORKERS

    def zero_body(r, _):
        for j in range(8):
            msgb[r, pl.ds(j * 16, 16)] = jnp.zeros((16,), jnp.float32)
        return 0

    for p in range(4):
        lax.fori_loop(0, CHUNK, zero_body, 0)
        for z in range(ROWS_PER_TILE // CHUNK):
            pltpu.sync_copy(
                msgb, otab.at[pl.ds(sid * ROWS_PER_TILE + z * CHUNK, CHUNK)])
        plsc.subcore_barrier()

        def issue(t, b):
            base = (wid + NWORKERS * t) * CHUNK
            pltpu.sync_copy(s_hbm.at[pl.ds(base, CHUNK)], sv)
            for g in range(CHUNK // 16):
                s4s[b][pl.ds(g * 16, 16)] = sv[pl.ds(g * 16, 16)] * 4 + p
            pltpu.async_copy(vrel4_hbm.at[s4s[b]], vrs[b], sms[b])

        def compute(t, b):
            base = (wid + NWORKERS * t) * CHUNK
            pltpu.sync_copy(d_hbm.at[pl.ds(base, CHUNK)], dv)
            pltpu.sync_copy(coef_hbm.at[pl.ds(base, CHUNK)], cfb)
            pltpu.make_async_copy(vrel4_hbm.at[s4s[b]], vrs[b], sms[b]).wait()
            vrows = vrs[b]

            def edge_body(e2, _2):
                for u in range(2):
                    e = e2 * 2 + u
                    crow = cfb[e, :]
                    c0 = crow[2 * p]
                    c1 = crow[2 * p + 1]
                    for j in range(8):
                        cc = c0 if j < 4 else c1
                        msgb[e, pl.ds(j * 16, 16)] = (
                            vrows[e, pl.ds(j * 16, 16)] * cc)
                return 0

            lax.fori_loop(0, CHUNK // 2, edge_body, 0)
            pltpu.sync_copy(msgb, otab.at[dv], add=True)

        issue(0, 0)
        issue(1, 1)

        def pair_body(t2, _):
            for b in range(2):
                t = t2 * 2 + b
                compute(t, b)

                @pl.when(t + 2 < MMAIN)
                def _():
                    issue(t + 2, b)
            return 0

        lax.fori_loop(0, MMAIN // 2, pair_body, 0)

        @pl.when(wid < MEXTRA)
        def _():
            t_tail = MMAIN * NWORKERS + wid
            base = t_tail * CHUNK
            pltpu.sync_copy(s_hbm.at[pl.ds(base, CHUNK)], sv)
            for g in range(CHUNK // 16):
                s4v0[pl.ds(g * 16, 16)] = sv[pl.ds(g * 16, 16)] * 4 + p
            pltpu.sync_copy(d_hbm.at[pl.ds(base, CHUNK)], dv)
            pltpu.sync_copy(coef_hbm.at[pl.ds(base, CHUNK)], cfb)
            pltpu.async_copy(vrel4_hbm.at[s4v0], vr0, sm0).wait()

            def edge_body(e2, _2):
                for u in range(2):
                    e = e2 * 2 + u
                    crow = cfb[e, :]
                    c0 = crow[2 * p]
                    c1 = crow[2 * p + 1]
                    for j in range(8):
                        cc = c0 if j < 4 else c1
                        msgb[e, pl.ds(j * 16, 16)] = (
                            vr0[e, pl.ds(j * 16, 16)] * cc)
                return 0

            lax.fori_loop(0, CHUNK // 2, edge_body, 0)
            pltpu.sync_copy(msgb, otab.at[dv], add=True)

        plsc.subcore_barrier()
        pltpu.sync_copy(
            otab.at[pl.ds(sid * ROWS_PER_TILE, ROWS_PER_TILE)],
            opart_hbm.at[pl.ds((c * 4 + p) * NPAD + sid * ROWS_PER_TILE,
                               ROWS_PER_TILE)])
        plsc.subcore_barrier()


def _edge_stage(q_dst, k_rel, v_rel, s_idx, d_idx, p_rel):
    pvec = jnp.zeros((16,), jnp.float32).at[:H].set(p_rel / np.sqrt(D))
    lg, wmax = _sc_logits(q_dst, k_rel, s_idx, d_idx, pvec)
    dpart = _sc_denom(lg, d_idx, wmax)
    coef = _sc_coef(lg, d_idx, wmax, dpart)
    vrel4 = v_rel.reshape(NPAD * 4, 128)
    return _sc_msg(vrel4, s_idx, d_idx, coef)


def kernel(x_author, x_paper, edge_index__author__writes__paper,
           edge_index__paper__rev_writes__author, params):
    ei = {_ekey(ET[0]): edge_index__author__writes__paper,
          _ekey(ET[1]): edge_index__paper__rev_writes__author}
    pad = ((0, NPAD - N), (0, 0))
    x = {'author': _matmul(jnp.pad(x_author, pad), params['in']['author']['w'],
                           params['in']['author']['b'], act_out='relu'),
         'paper': _matmul(jnp.pad(x_paper, pad), params['in']['paper']['w'],
                          params['in']['paper']['b'], act_out='relu')}
    for layer in params['layers']:
        q = {t: _matmul(x[t], layer['q'][t]['w'], layer['q'][t]['b']) for t in NT}
        krel = {}
        vrel = {}
        for et in ET:
            src, _, dst = et
            r = layer['rel'][_ekey(et)]
            wk, bk = _combine_rel(layer['k'][src]['w'], layer['k'][src]['b'], r['a_rel'])
            wv, bv = _combine_rel(layer['v'][src]['w'], layer['v'][src]['b'], r['m_rel'])
            krel[_ekey(et)] = _matmul(x[src], wk, bk)
            vrel[_ekey(et)] = _matmul(x[src], wv, bv)
        out = {}
        for et in ET:
            src, _, dst = et
            ek = _ekey(et)
            e = ei[ek]
            out[dst] = _edge_stage(q[dst], krel[ek], vrel[ek], e[0], e[1],
                                   layer['rel'][ek]['p_rel'])
        newx = {}
        for t in NT:
            beta = jax.nn.sigmoid(layer['skip'][t])
            w = layer['a'][t]['w'] * beta
            b = layer['a'][t]['b'] * beta
            newx[t] = _matmul_agg(out[t], w, b, res=(1.0 - beta) * x[t])
        x = newx
    return _matmul(x['author'], params['out']['w'], params['out']['b'])[:N]
